# baseline (speedup 1.0000x reference)
import functools

import jax
import jax.numpy as jnp
from jax import lax
from jax.experimental import pallas as pl
from jax.experimental.pallas import tpu as pltpu
from jax.experimental.pallas import tpu_sc as plsc

_LANES = 16
_IDXW = 128
_HR = 8


@functools.lru_cache(maxsize=None)
def _build(V1, D, BATCH, H, NC, NS):
    NW = NC * NS
    A = D // _HR
    HB = H // _HR
    CB = BATCH // _IDXW
    n_units = HB * CB
    u_per_w = n_units // NW
    assert D % _HR == 0 and H % _HR == 0 and BATCH % _IDXW == 0
    assert n_units % NW == 0 and u_per_w % 2 == 0

    mesh = plsc.VectorSubcoreMesh(core_axis_name="c", subcore_axis_name="s")

    @functools.partial(
        pl.kernel,
        mesh=mesh,
        compiler_params=pltpu.CompilerParams(
            use_tc_tiling_on_sc=False, needs_layout_passes=False),
        out_type=jax.ShapeDtypeStruct((H, A, CB, _HR, _IDXW), jnp.float32),
        scratch_types=[
            pltpu.VMEM((2, _HR, _IDXW), jnp.int32),
            pltpu.VMEM((_HR * _IDXW, D), jnp.float32),
            pltpu.VMEM((2, _HR, A, 1, _HR, _IDXW), jnp.float32),
            pltpu.SemaphoreType.DMA,
            pltpu.SemaphoreType.DMA,
            pltpu.SemaphoreType.DMA,
            pltpu.SemaphoreType.DMA,
            pltpu.SemaphoreType.DMA,
        ],
    )
    def emb(idx_hbm, table_hbm, out_hbm, idx_v, rows_v, val_v, gat_sem,
            i_sem0, i_sem1, o_sem0, o_sem1):
        wid = lax.axis_index("s") * NC + lax.axis_index("c")
        u0 = wid * u_per_w
        i_sems = (i_sem0, i_sem1)
        o_sems = (o_sem0, o_sem1)

        def unit_coords(u):
            return u // CB, lax.rem(u, CB)

        for p in range(2):
            hb, bb = unit_coords(u0 + p)
            pltpu.async_copy(idx_hbm.at[hb, bb], idx_v.at[p], i_sems[p])

        def outer(t, carry):
            for p in range(2):
                u = u0 + 2 * t + p
                hb, bb = unit_coords(u)

                pltpu.make_async_copy(
                    idx_hbm.at[0, 0], idx_v.at[p], i_sems[p]).wait()

                for j in range(_HR):
                    pltpu.async_copy(
                        table_hbm.at[idx_v.at[p].at[j]],
                        rows_v.at[pl.ds(j * _IDXW, _IDXW)],
                        gat_sem,
                    )

                @pl.when(t > 0)
                def _wait_store(p=p):
                    pltpu.make_async_copy(
                        val_v.at[p],
                        out_hbm.at[pl.ds(0, _HR), :, pl.ds(0, 1)],
                        o_sems[p]).wait()

                for j in range(_HR):
                    pltpu.make_async_copy(
                        table_hbm.at[idx_v.at[p].at[0]],
                        rows_v.at[pl.ds(0, _IDXW)],
                        gat_sem).wait()

                def block(j, carry):
                    base = j * _IDXW
                    masks = []
                    for tt in range(_IDXW // _LANES):
                        vi = idx_v[p, j, pl.ds(tt * _LANES, _LANES)]
                        masks.append(
                            jnp.where(vi == 0, jnp.float32(0), jnp.float32(1)))
                    for a in range(A):
                        for r in range(_HR):
                            col = jnp.zeros((_LANES,), jnp.int32) + (a * _HR + r)
                            for tt in range(_IDXW // _LANES):
                                rows = base + tt * _LANES + lax.iota(
                                    jnp.int32, _LANES)
                                g = plsc.load_gather(rows_v, [rows, col])
                                val_v[p, j, a, 0, r,
                                      pl.ds(tt * _LANES, _LANES)] = g * masks[tt]
                    return carry

                lax.fori_loop(0, _HR, block, 0)

                @pl.when(2 * t + p + 2 < u_per_w)
                def _prefetch(p=p, u=u):
                    hb2, bb2 = unit_coords(u + 2)
                    pltpu.async_copy(idx_hbm.at[hb2, bb2], idx_v.at[p],
                                     i_sems[p])

                pltpu.async_copy(
                    val_v.at[p],
                    out_hbm.at[pl.ds(hb * _HR, _HR), :, pl.ds(bb, 1)],
                    o_sems[p])
            return carry

        lax.fori_loop(0, u_per_w // 2, outer, 0)

        for p in range(2):
            pltpu.make_async_copy(
                val_v.at[p],
                out_hbm.at[pl.ds(0, _HR), :, pl.ds(0, 1)],
                o_sems[p]).wait()

    return emb


def kernel(feature_ids, table):
    batch, hist = feature_ids.shape
    V1, D = table.shape
    B = batch * hist
    ids32 = feature_ids.astype(jnp.int32)
    i4 = ids32.reshape(batch // _IDXW, _IDXW, hist // _HR, _HR)
    i4 = i4.transpose(2, 0, 3, 1)
    info = plsc.get_sparse_core_info()
    emb = _build(V1, D, batch, hist, info.num_cores, info.num_subcores)
    out5 = emb(i4, table)
    out = out5.transpose(2, 4, 0, 1, 3).reshape(batch, hist, D)
    return out
